# baseline (speedup 1.0000x reference)
import functools

import jax
import jax.numpy as jnp
from jax import lax
from jax.experimental import pallas as pl
from jax.experimental.pallas import tpu as pltpu
from jax.experimental.pallas import tpu_sc as plsc

K = 16
MARGIN = 1.0
N = 10000
D = 256
A = 2048
BA = 128


def _sc_gather_anchors(out1, out2, anchor1, anchor2):
    info = plsc.get_sparse_core_info()
    nw = info.num_cores * info.num_subcores
    bpw = A // nw
    mesh = plsc.VectorSubcoreMesh(core_axis_name="c", subcore_axis_name="s")

    @functools.partial(
        pl.kernel,
        mesh=mesh,
        out_type=[
            jax.ShapeDtypeStruct((A, D), jnp.float32),
            jax.ShapeDtypeStruct((A, D), jnp.float32),
        ],
        scratch_types=[
            pltpu.VMEM((bpw,), jnp.int32),
            pltpu.VMEM((bpw, D), jnp.float32),
            pltpu.SemaphoreType.DMA,
        ],
    )
    def gather_kernel(t1_hbm, t2_hbm, i1_hbm, i2_hbm, o1_hbm, o2_hbm,
                      idx_v, rows_v, sem):
        wid = lax.axis_index("s") * info.num_cores + lax.axis_index("c")
        base = wid * bpw
        pltpu.sync_copy(i1_hbm.at[pl.ds(base, bpw)], idx_v)
        pltpu.async_copy(t1_hbm.at[idx_v], rows_v, sem).wait()
        pltpu.sync_copy(rows_v, o1_hbm.at[pl.ds(base, bpw)])
        pltpu.sync_copy(i2_hbm.at[pl.ds(base, bpw)], idx_v)
        pltpu.async_copy(t2_hbm.at[idx_v], rows_v, sem).wait()
        pltpu.sync_copy(rows_v, o2_hbm.at[pl.ds(base, bpw)])

    return gather_kernel(out1, out2, anchor1.astype(jnp.int32),
                         anchor2.astype(jnp.int32))


def _tc_loss_body(t2_ref, t1_ref, ae1_ref, ae2_ref, out_ref,
                  simt_ref, rinv2_ref, rinv1_ref):
    a_blk = pl.program_id(0)

    @pl.when(a_blk == 0)
    def _init():
        for tref, rref in ((t2_ref, rinv2_ref), (t1_ref, rinv1_ref)):
            t32 = tref[...].astype(jnp.float32)
            rref[...] = lax.rsqrt(jnp.sum(t32 * t32, axis=1, keepdims=True))
        out_ref[0, 0] = 0.0

    ae1 = ae1_ref[...]
    ae2 = ae2_ref[...]
    n1sq = jnp.sum(ae1 * ae1, axis=1, keepdims=True)
    n2sq = jnp.sum(ae2 * ae2, axis=1, keepdims=True)
    ae1n = ae1 * lax.rsqrt(n1sq)
    ae2n = ae2 * lax.rsqrt(n2sq)

    c = 1.0 - jnp.sum(ae1n * ae2n, axis=1)

    neg = jnp.float32(-3.0)

    total = jnp.float32(0.0)
    for side in range(2):
        tbl = t2_ref[...] if side == 0 else t1_ref[...]
        rinv = rinv2_ref[...] if side == 0 else rinv1_ref[...]
        anc = (ae1n if side == 0 else ae2n).astype(jnp.bfloat16)
        simt = lax.dot_general(
            tbl, anc, (((1,), (1,)), ((), ())),
            preferred_element_type=jnp.float32)
        s = simt * rinv
        simt_ref[...] = s
        m = jnp.max(s, axis=0)
        acc = jnp.maximum(c + m, 0.0)
        for _ in range(K - 1):
            s = simt_ref[...]
            s = jnp.where(s == m[None, :], neg, s)
            simt_ref[...] = s
            m = jnp.max(s, axis=0)
            acc = acc + jnp.maximum(c + m, 0.0)
        total = total + jnp.sum(acc)

    out_ref[0, 0] += total / (A * K)


def _tc_loss(out2b, out1b, ae1, ae2):
    grid = (A // BA,)
    return pl.pallas_call(
        _tc_loss_body,
        grid=grid,
        in_specs=[
            pl.BlockSpec((N, D), lambda a: (0, 0)),
            pl.BlockSpec((N, D), lambda a: (0, 0)),
            pl.BlockSpec((BA, D), lambda a: (a, 0)),
            pl.BlockSpec((BA, D), lambda a: (a, 0)),
        ],
        out_specs=pl.BlockSpec(memory_space=pltpu.SMEM),
        out_shape=jax.ShapeDtypeStruct((1, 1), jnp.float32),
        scratch_shapes=[
            pltpu.VMEM((N, BA), jnp.float32),
            pltpu.VMEM((N, 1), jnp.float32),
            pltpu.VMEM((N, 1), jnp.float32),
        ],
    )(out2b, out1b, ae1, ae2)


def kernel(out1, out2, anchor1, anchor2):
    ae1, ae2 = _sc_gather_anchors(out1, out2, anchor1, anchor2)
    loss = _tc_loss(out2.astype(jnp.bfloat16), out1.astype(jnp.bfloat16),
                    ae1, ae2)
    return loss[0, 0]


# speedup vs baseline: 1.1761x; 1.1761x over previous
import functools

import jax
import jax.numpy as jnp
from jax import lax
from jax.experimental import pallas as pl
from jax.experimental.pallas import tpu as pltpu
from jax.experimental.pallas import tpu_sc as plsc

K = 16
MARGIN = 1.0
N = 10000
NPAD = 10240
D = 256
A = 2048
BA = 256


def _sc_gather_anchors(out1, out2, anchor1, anchor2):
    info = plsc.get_sparse_core_info()
    nw = info.num_cores * info.num_subcores
    bpw = A // nw
    mesh = plsc.VectorSubcoreMesh(core_axis_name="c", subcore_axis_name="s")

    @functools.partial(
        pl.kernel,
        mesh=mesh,
        out_type=[
            jax.ShapeDtypeStruct((A, D), jnp.float32),
            jax.ShapeDtypeStruct((A, D), jnp.float32),
        ],
        scratch_types=[
            pltpu.VMEM((bpw,), jnp.int32),
            pltpu.VMEM((bpw, D), jnp.float32),
            pltpu.SemaphoreType.DMA,
        ],
    )
    def gather_kernel(t1_hbm, t2_hbm, i1_hbm, i2_hbm, o1_hbm, o2_hbm,
                      idx_v, rows_v, sem):
        wid = lax.axis_index("s") * info.num_cores + lax.axis_index("c")
        base = wid * bpw
        pltpu.sync_copy(i1_hbm.at[pl.ds(base, bpw)], idx_v)
        pltpu.async_copy(t1_hbm.at[idx_v], rows_v, sem).wait()
        pltpu.sync_copy(rows_v, o1_hbm.at[pl.ds(base, bpw)])
        pltpu.sync_copy(i2_hbm.at[pl.ds(base, bpw)], idx_v)
        pltpu.async_copy(t2_hbm.at[idx_v], rows_v, sem).wait()
        pltpu.sync_copy(rows_v, o2_hbm.at[pl.ds(base, bpw)])

    return gather_kernel(out1, out2, anchor1.astype(jnp.int32),
                         anchor2.astype(jnp.int32))


def _tc_loss_body(out2p_ref, out1p_ref, ae1_ref, ae2_ref, out_ref, simt_ref):
    a_blk = pl.program_id(0)

    ae1 = ae1_ref[...]
    ae2 = ae2_ref[...]
    n1sq = jnp.sum(ae1 * ae1, axis=1, keepdims=True)
    n2sq = jnp.sum(ae2 * ae2, axis=1, keepdims=True)
    ae1n = ae1 * lax.rsqrt(n1sq)
    ae2n = ae2 * lax.rsqrt(n2sq)

    c = 1.0 - jnp.sum(ae1n * ae2n, axis=1)

    row_ids = lax.broadcasted_iota(jnp.int32, (NPAD, 1), 0)
    neg = jnp.float32(-1e30)

    total = jnp.float32(0.0)
    for side in range(2):
        tbl = out2p_ref[...] if side == 0 else out1p_ref[...]
        anc = ae1n if side == 0 else ae2n
        tsq = jnp.sum(tbl * tbl, axis=1, keepdims=True)
        simt = lax.dot_general(
            tbl, anc, (((1,), (1,)), ((), ())),
            preferred_element_type=jnp.float32)
        simt = simt * lax.rsqrt(tsq)
        simt = jnp.where(row_ids < N, simt, neg)
        simt_ref[...] = simt
        acc = jnp.zeros((BA,), jnp.float32)
        for _ in range(K):
            s = simt_ref[...]
            m = jnp.max(s, axis=0)
            acc = acc + jnp.maximum(c + m, 0.0)
            simt_ref[...] = jnp.where(s == m[None, :], neg, s)
        total = total + jnp.sum(acc)

    @pl.when(a_blk == 0)
    def _init():
        out_ref[0, 0] = 0.0

    out_ref[0, 0] += total / (A * K)


def _tc_loss(out2p, out1p, ae1, ae2):
    grid = (A // BA,)
    return pl.pallas_call(
        _tc_loss_body,
        grid=grid,
        in_specs=[
            pl.BlockSpec((NPAD, D), lambda a: (0, 0)),
            pl.BlockSpec((NPAD, D), lambda a: (0, 0)),
            pl.BlockSpec((BA, D), lambda a: (a, 0)),
            pl.BlockSpec((BA, D), lambda a: (a, 0)),
        ],
        out_specs=pl.BlockSpec(memory_space=pltpu.SMEM),
        out_shape=jax.ShapeDtypeStruct((1, 1), jnp.float32),
        scratch_shapes=[pltpu.VMEM((NPAD, BA), jnp.float32)],
    )(out2p, out1p, ae1, ae2)


def kernel(out1, out2, anchor1, anchor2):
    ae1, ae2 = _sc_gather_anchors(out1, out2, anchor1, anchor2)
    pad = ((0, NPAD - N), (0, 0))
    out1p = jnp.pad(out1, pad)
    out2p = jnp.pad(out2, pad)
    loss = _tc_loss(out2p, out1p, ae1, ae2)
    return loss[0, 0]


# speedup vs baseline: 1.4781x; 1.2567x over previous
import functools

import jax
import jax.numpy as jnp
from jax import lax
from jax.experimental import pallas as pl
from jax.experimental.pallas import tpu as pltpu
from jax.experimental.pallas import tpu_sc as plsc

K = 16
MARGIN = 1.0
N = 10000
NPAD = 10240
D = 256
A = 2048
BA = 128


def _sc_gather_anchors(out1, out2, anchor1, anchor2):
    info = plsc.get_sparse_core_info()
    nw = info.num_cores * info.num_subcores
    bpw = A // nw
    mesh = plsc.VectorSubcoreMesh(core_axis_name="c", subcore_axis_name="s")

    @functools.partial(
        pl.kernel,
        mesh=mesh,
        out_type=[
            jax.ShapeDtypeStruct((A, D), jnp.float32),
            jax.ShapeDtypeStruct((A, D), jnp.float32),
        ],
        scratch_types=[
            pltpu.VMEM((bpw,), jnp.int32),
            pltpu.VMEM((bpw, D), jnp.float32),
            pltpu.SemaphoreType.DMA,
        ],
    )
    def gather_kernel(t1_hbm, t2_hbm, i1_hbm, i2_hbm, o1_hbm, o2_hbm,
                      idx_v, rows_v, sem):
        wid = lax.axis_index("s") * info.num_cores + lax.axis_index("c")
        base = wid * bpw
        pltpu.sync_copy(i1_hbm.at[pl.ds(base, bpw)], idx_v)
        pltpu.async_copy(t1_hbm.at[idx_v], rows_v, sem).wait()
        pltpu.sync_copy(rows_v, o1_hbm.at[pl.ds(base, bpw)])
        pltpu.sync_copy(i2_hbm.at[pl.ds(base, bpw)], idx_v)
        pltpu.async_copy(t2_hbm.at[idx_v], rows_v, sem).wait()
        pltpu.sync_copy(rows_v, o2_hbm.at[pl.ds(base, bpw)])

    return gather_kernel(out1, out2, anchor1.astype(jnp.int32),
                         anchor2.astype(jnp.int32))


def _tc_loss_body(out2p_ref, out1p_ref, ae1_ref, ae2_ref, out_ref, simt_ref):
    a_blk = pl.program_id(0)

    ae1 = ae1_ref[...]
    ae2 = ae2_ref[...]
    n1sq = jnp.sum(ae1 * ae1, axis=1, keepdims=True)
    n2sq = jnp.sum(ae2 * ae2, axis=1, keepdims=True)
    ae1n = ae1 * lax.rsqrt(n1sq)
    ae2n = ae2 * lax.rsqrt(n2sq)

    c = 1.0 - jnp.sum(ae1n * ae2n, axis=1)

    row_ids = lax.broadcasted_iota(jnp.int32, (NPAD, 1), 0)
    neg = jnp.float32(-1e30)

    total = jnp.float32(0.0)
    for side in range(2):
        tbl = out2p_ref[...] if side == 0 else out1p_ref[...]
        anc = ae1n if side == 0 else ae2n
        tsq = jnp.sum(tbl * tbl, axis=1, keepdims=True)
        simt = lax.dot_general(
            tbl, anc, (((1,), (1,)), ((), ())),
            preferred_element_type=jnp.float32)
        simt = simt * lax.rsqrt(tsq)
        simt = jnp.where(row_ids < N, simt, neg)
        simt_ref[...] = simt
        acc = jnp.zeros((BA,), jnp.float32)
        for _ in range(K):
            s = simt_ref[...]
            m = jnp.max(s, axis=0)
            acc = acc + jnp.maximum(c + m, 0.0)
            simt_ref[...] = jnp.where(s == m[None, :], neg, s)
        total = total + jnp.sum(acc)

    @pl.when(a_blk == 0)
    def _init():
        out_ref[0, 0] = 0.0

    out_ref[0, 0] += total / (A * K)


def _tc_loss(out2p, out1p, ae1, ae2):
    grid = (A // BA,)
    return pl.pallas_call(
        _tc_loss_body,
        grid=grid,
        in_specs=[
            pl.BlockSpec((NPAD, D), lambda a: (0, 0)),
            pl.BlockSpec((NPAD, D), lambda a: (0, 0)),
            pl.BlockSpec((BA, D), lambda a: (a, 0)),
            pl.BlockSpec((BA, D), lambda a: (a, 0)),
        ],
        out_specs=pl.BlockSpec(memory_space=pltpu.SMEM),
        out_shape=jax.ShapeDtypeStruct((1, 1), jnp.float32),
        scratch_shapes=[pltpu.VMEM((NPAD, BA), jnp.float32)],
    )(out2p, out1p, ae1, ae2)


def kernel(out1, out2, anchor1, anchor2):
    ae1, ae2 = _sc_gather_anchors(out1, out2, anchor1, anchor2)
    pad = ((0, NPAD - N), (0, 0))
    out1p = jnp.pad(out1, pad)
    out2p = jnp.pad(out2, pad)
    loss = _tc_loss(out2p, out1p, ae1, ae2)
    return loss[0, 0]
